# Initial kernel scaffold; baseline (speedup 1.0000x reference)
import jax
import jax.numpy as jnp
from jax.experimental import pallas as pl


def kernel(x, edge_index, batch, W1, b1, W2, b2, W3, b3, W4, b4, Wl, bl):
    raise NotImplementedError("write your pallas kernel here")


# speedup vs baseline: 23.3210x; 23.3210x over previous
import functools

import jax
import jax.numpy as jnp
import numpy as np
from jax import lax
from jax.experimental import pallas as pl
from jax.experimental.pallas import tpu as pltpu
from jax.experimental.pallas import tpu_sc as plsc

N_NODES = 50000
N_EDGES = 1600000
N_GRAPHS = 128

NT = 53248
EP = 1605632
NBLK = EP // 128
ROWS_PER_TILE = NT // 16
ZR = 208
CB = 28

_SC_PARAMS = pltpu.CompilerParams(needs_layout_passes=False,
                                  use_tc_tiling_on_sc=False)


def _wid(c, s):
    return c * 16 + s


def _make_deg():
    mesh = plsc.VectorSubcoreMesh(core_axis_name="c", subcore_axis_name="s")
    EPT = EP // 32
    CH = EPT // 8

    @functools.partial(
        pl.kernel, mesh=mesh,
        out_type=jax.ShapeDtypeStruct((2, NT), jnp.float32),
        compiler_params=_SC_PARAMS,
        scratch_types=[
            pltpu.VMEM((NT,), jnp.float32),
            pltpu.VMEM((CH,), jnp.int32),
            pltpu.VMEM((ROWS_PER_TILE,), jnp.float32),
            pltpu.VMEM((ROWS_PER_TILE,), jnp.float32),
            pltpu.VMEM_SHARED((16, NT), jnp.float32),
        ],
    )
    def deg_kernel(dst_flat, out, hist, chunk, tmp, acc, sh):
        c = lax.axis_index("c")
        s = lax.axis_index("s")
        w = _wid(c, s)
        zero16 = jnp.zeros((16,), jnp.float32)
        ones16 = jnp.full((16,), 1.0, jnp.float32)

        def zr(i, _):
            hist[pl.ds(i * 16, 16)] = zero16
            return _
        lax.fori_loop(0, NT // 16, zr, None)

        e0 = w * EPT
        for ch in range(8):
            pltpu.sync_copy(dst_flat.at[pl.ds(e0 + ch * CH, CH)], chunk)

            def body(j, _):
                idx = chunk[pl.ds(j * 16, 16)]
                plsc.addupdate_scatter(hist, [idx], ones16)
                return _
            lax.fori_loop(0, CH // 16, body, None)

        pltpu.sync_copy(hist, sh.at[s])
        plsc.subcore_barrier()

        r0 = s * ROWS_PER_TILE
        pltpu.sync_copy(sh.at[0, pl.ds(r0, ROWS_PER_TILE)], acc)
        for t in range(1, 16):
            pltpu.sync_copy(sh.at[t, pl.ds(r0, ROWS_PER_TILE)], tmp)

            def addb(k, _):
                sl = pl.ds(k * 16, 16)
                acc[sl] = acc[sl] + tmp[sl]
                return _
            lax.fori_loop(0, ROWS_PER_TILE // 16, addb, None)
        pltpu.sync_copy(acc, out.at[c, pl.ds(r0, ROWS_PER_TILE)])

    return deg_kernel


def _make_agg(F, feat_split):
    mesh = plsc.VectorSubcoreMesh(core_axis_name="c", subcore_axis_name="s")
    blocks_per_tile = (NBLK // 16) if feat_split else (NBLK // 32)
    n_chunks = blocks_per_tile // CB

    if feat_split:
        out_type = [jax.ShapeDtypeStruct((NT, F), jnp.float32),
                    jax.ShapeDtypeStruct((NT, F), jnp.float32)]
    else:
        out_type = jax.ShapeDtypeStruct((2, NT, F), jnp.float32)

    scratch = [
        pltpu.VMEM((CB, 128), jnp.int32),
        pltpu.VMEM((CB, 128), jnp.int32),
        pltpu.VMEM((128, F), jnp.float32),
        pltpu.VMEM((ZR, F), jnp.float32),
        pltpu.VMEM_SHARED((NT, F), jnp.float32),
        pltpu.SemaphoreType.DMA,
    ]

    def body(refs):
        if feat_split:
            (ta, tb, src2d, dst2d, oa, ob, sbuf, dbuf, rows, zbuf, acc,
             sem) = refs
        else:
            (table, src2d, dst2d, out, sbuf, dbuf, rows, zbuf, acc,
             sem) = refs
        c = lax.axis_index("c")
        s = lax.axis_index("s")
        zero16 = jnp.zeros((16,), jnp.float32)

        def zr(r, _):
            for cc in range(F // 16):
                zbuf[r, pl.ds(cc * 16, 16)] = zero16
            return _
        lax.fori_loop(0, ZR, zr, None)
        r0 = s * ROWS_PER_TILE
        for k in range(16):
            pltpu.sync_copy(zbuf, acc.at[pl.ds(r0 + k * ZR, ZR), :])
        plsc.subcore_barrier()

        def edge_pass(table_ref, blk0):
            def chunk_body(ci, _):
                base = blk0 + ci * CB
                pltpu.sync_copy(src2d.at[pl.ds(base, CB)], sbuf)
                pltpu.sync_copy(dst2d.at[pl.ds(base, CB)], dbuf)

                def blk(j, _):
                    pltpu.async_copy(table_ref.at[sbuf.at[j]], rows,
                                     sem).wait()
                    pltpu.sync_copy(rows, acc.at[dbuf.at[j]], add=True)
                    return _
                lax.fori_loop(0, CB, blk, None)
                return _
            lax.fori_loop(0, n_chunks, chunk_body, None)

        def writeout(out_ref):
            for k in range(16):
                pltpu.sync_copy(acc.at[pl.ds(r0 + k * ZR, ZR), :], zbuf)
                pltpu.sync_copy(zbuf, out_ref.at[pl.ds(r0 + k * ZR, ZR), :])

        if feat_split:
            blk0 = s * blocks_per_tile

            @pl.when(c == 0)
            def _():
                edge_pass(ta, blk0)

            @pl.when(c == 1)
            def _():
                edge_pass(tb, blk0)
            plsc.subcore_barrier()

            @pl.when(c == 0)
            def _():
                writeout(oa)

            @pl.when(c == 1)
            def _():
                writeout(ob)
        else:
            blk0 = _wid(c, s) * blocks_per_tile
            edge_pass(table, blk0)
            plsc.subcore_barrier()
            for k in range(16):
                pltpu.sync_copy(acc.at[pl.ds(r0 + k * ZR, ZR), :], zbuf)
                pltpu.sync_copy(zbuf, out.at[c, pl.ds(r0 + k * ZR, ZR), :])

    if feat_split:
        @functools.partial(pl.kernel, mesh=mesh, out_type=out_type,
                           compiler_params=_SC_PARAMS,
                           scratch_types=scratch)
        def agg_kernel(ta, tb, src2d, dst2d, oa, ob, sbuf, dbuf, rows,
                       zbuf, acc, sem):
            body((ta, tb, src2d, dst2d, oa, ob, sbuf, dbuf, rows, zbuf,
                  acc, sem))
    else:
        @functools.partial(pl.kernel, mesh=mesh, out_type=out_type,
                           compiler_params=_SC_PARAMS,
                           scratch_types=scratch)
        def agg_kernel(table, src2d, dst2d, out, sbuf, dbuf, rows, zbuf,
                       acc, sem):
            body((table, src2d, dst2d, out, sbuf, dbuf, rows, zbuf, acc,
                  sem))

    return agg_kernel


def _make_readout():
    mesh = plsc.VectorSubcoreMesh(core_axis_name="c", subcore_axis_name="s")
    BPT = (NT // 128) // 32

    @functools.partial(
        pl.kernel, mesh=mesh,
        out_type=jax.ShapeDtypeStruct((2, N_GRAPHS, 16), jnp.float32),
        compiler_params=_SC_PARAMS,
        scratch_types=[
            pltpu.VMEM((BPT, 128), jnp.int32),
            pltpu.VMEM((128, 16), jnp.float32),
            pltpu.VMEM((N_GRAPHS, 16), jnp.float32),
            pltpu.VMEM((16,), jnp.float32),
            pltpu.VMEM_SHARED((N_GRAPHS, 16), jnp.float32),
            pltpu.SemaphoreType.DMA,
        ],
    )
    def readout_kernel(rows_hbm, batch2d, out, bbuf, rbuf, obuf, z16, shr,
                       sem):
        c = lax.axis_index("c")
        s = lax.axis_index("s")
        z16[pl.ds(0, 16)] = jnp.zeros((16,), jnp.float32)

        def zrow(i, _):
            pltpu.sync_copy(z16, shr.at[i])
            return _
        lax.fori_loop(s * 8, (s + 1) * 8, zrow, None)
        plsc.subcore_barrier()

        blk0 = _wid(c, s) * BPT
        pltpu.sync_copy(batch2d.at[pl.ds(blk0, BPT)], bbuf)

        def body(j, _):
            pltpu.sync_copy(
                rows_hbm.at[pl.ds((blk0 + j) * 128, 128), :], rbuf)
            pltpu.sync_copy(rbuf, shr.at[bbuf.at[j]], add=True)
            return _
        lax.fori_loop(0, BPT, body, None)
        plsc.subcore_barrier()

        @pl.when(s == 0)
        def _():
            pltpu.sync_copy(shr, obuf)
            pltpu.sync_copy(obuf, out.at[c])

    return readout_kernel


def _valid_mask(rows):
    base = pl.program_id(0) * rows
    ii = lax.broadcasted_iota(jnp.int32, (rows, 1), 0) + base
    return (ii < N_NODES).astype(jnp.float32)


def _dinv(p0, p1, v):
    return lax.rsqrt(p0 + p1 + 1.0) * v


BR = 1024


def _tc_prep(p0, p1, xpad):
    def body(p0_r, p1_r, x_r, o_r):
        v = _valid_mask(BR)
        di = _dinv(p0_r[...], p1_r[...], v)
        o_r[...] = x_r[...] * di

    return pl.pallas_call(
        body,
        grid=(NT // BR,),
        in_specs=[
            pl.BlockSpec((BR, 1), lambda i: (i, 0)),
            pl.BlockSpec((BR, 1), lambda i: (i, 0)),
            pl.BlockSpec((BR, 16), lambda i: (i, 0)),
        ],
        out_specs=pl.BlockSpec((BR, 16), lambda i: (i, 0)),
        out_shape=jax.ShapeDtypeStruct((NT, 16), jnp.float32),
    )(p0, p1, xpad)


def _tc_layer(s0, s1, hp, p0, p1, W, b, split_out):
    Fi = W.shape[0]
    Fo = W.shape[1]

    def body(s0_r, s1_r, hp_r, p0_r, p1_r, w_r, b_r, *outs):
        v = _valid_mask(BR)
        di = _dinv(p0_r[...], p1_r[...], v)
        agg = (s0_r[...] + s1_r[...] + hp_r[...]) * di
        h = jnp.maximum(
            jnp.dot(agg, w_r[...], preferred_element_type=jnp.float32)
            + b_r[...], 0.0)
        hpn = h * di
        if split_out:
            outs[0][...] = hpn[:, :Fo // 2]
            outs[1][...] = hpn[:, Fo // 2:]
        else:
            outs[0][...] = hpn

    if split_out:
        out_shape = [jax.ShapeDtypeStruct((NT, Fo // 2), jnp.float32),
                     jax.ShapeDtypeStruct((NT, Fo // 2), jnp.float32)]
        out_specs = [pl.BlockSpec((BR, Fo // 2), lambda i: (i, 0)),
                     pl.BlockSpec((BR, Fo // 2), lambda i: (i, 0))]
    else:
        out_shape = jax.ShapeDtypeStruct((NT, Fo), jnp.float32)
        out_specs = pl.BlockSpec((BR, Fo), lambda i: (i, 0))

    return pl.pallas_call(
        body,
        grid=(NT // BR,),
        in_specs=[
            pl.BlockSpec((BR, Fi), lambda i: (i, 0)),
            pl.BlockSpec((BR, Fi), lambda i: (i, 0)),
            pl.BlockSpec((BR, Fi), lambda i: (i, 0)),
            pl.BlockSpec((BR, 1), lambda i: (i, 0)),
            pl.BlockSpec((BR, 1), lambda i: (i, 0)),
            pl.BlockSpec((Fi, Fo), lambda i: (0, 0)),
            pl.BlockSpec((1, Fo), lambda i: (0, 0)),
        ],
        out_specs=out_specs,
        out_shape=out_shape,
    )(s0, s1, hp, p0, p1, W, b)


def _tc_layer4(s4a, s4b, hp3a, hp3b, p0, p1, W4, b4, Wl, bl):
    def body(sa_r, sb_r, ha_r, hb_r, p0_r, p1_r, w4_r, b4_r, wl_r, bl_r,
             o_r):
        v = _valid_mask(BR)
        di = _dinv(p0_r[...], p1_r[...], v)
        agg = jnp.concatenate(
            [sa_r[...] + ha_r[...], sb_r[...] + hb_r[...]], axis=1) * di
        h4 = jnp.maximum(
            jnp.dot(agg, w4_r[...], preferred_element_type=jnp.float32)
            + b4_r[...], 0.0)
        head = jnp.dot(h4, wl_r[...], preferred_element_type=jnp.float32) \
            + bl_r[...]
        o_r[...] = head * v

    return pl.pallas_call(
        body,
        grid=(NT // BR,),
        in_specs=[
            pl.BlockSpec((BR, 32), lambda i: (i, 0)),
            pl.BlockSpec((BR, 32), lambda i: (i, 0)),
            pl.BlockSpec((BR, 32), lambda i: (i, 0)),
            pl.BlockSpec((BR, 32), lambda i: (i, 0)),
            pl.BlockSpec((BR, 1), lambda i: (i, 0)),
            pl.BlockSpec((BR, 1), lambda i: (i, 0)),
            pl.BlockSpec((64, 64), lambda i: (0, 0)),
            pl.BlockSpec((1, 64), lambda i: (0, 0)),
            pl.BlockSpec((64, 16), lambda i: (0, 0)),
            pl.BlockSpec((1, 16), lambda i: (0, 0)),
        ],
        out_specs=pl.BlockSpec((BR, 16), lambda i: (i, 0)),
        out_shape=jax.ShapeDtypeStruct((NT, 16), jnp.float32),
    )(s4a, s4b, hp3a, hp3b, p0, p1, W4, b4, Wl, bl)


def _tc_final(r):
    def body(r_ref, o_ref):
        sums = r_ref[0] + r_ref[1]
        cnt = jnp.maximum(sums[:, 10:11], 1.0)
        mean = sums[:, :10] / cnt
        mx = jnp.max(mean, axis=1, keepdims=True)
        lse = jnp.log(jnp.sum(jnp.exp(mean - mx), axis=1, keepdims=True))
        o_ref[...] = mean - mx - lse

    return pl.pallas_call(
        body,
        out_shape=jax.ShapeDtypeStruct((N_GRAPHS, 10), jnp.float32),
    )(r)


_deg_kernel = _make_deg()
_agg16 = _make_agg(16, feat_split=False)
_agg32 = _make_agg(32, feat_split=False)
_agg32f = _make_agg(32, feat_split=True)
_readout = _make_readout()


def kernel(x, edge_index, batch, W1, b1, W2, b2, W3, b3, W4, b4, Wl, bl):
    i32 = jnp.int32
    f32 = jnp.float32
    src = edge_index[0]
    dst = edge_index[1]
    pad_e = jnp.full((EP - N_EDGES,), N_NODES, i32)
    src_f = jnp.concatenate([src, pad_e])
    dst_f = jnp.concatenate([dst, pad_e])
    src2d = src_f.reshape(NBLK, 128)
    dst2d = dst_f.reshape(NBLK, 128)
    batch2d = jnp.concatenate(
        [batch, jnp.zeros((NT - N_NODES,), i32)]).reshape(NT // 128, 128)

    xpad = jnp.zeros((NT, 16), f32).at[:N_NODES, :3].set(x)
    W1p = jnp.zeros((16, 16), f32).at[:3].set(W1)
    Wlp = jnp.zeros((64, 16), f32).at[:, :10].set(Wl)
    blp = jnp.zeros((16,), f32).at[:10].set(bl).at[10].set(0.0)
    blp = blp.at[10].set(1.0)

    degp = _deg_kernel(dst_f)
    p0 = degp[0].reshape(NT, 1)
    p1 = degp[1].reshape(NT, 1)

    hp0 = _tc_prep(p0, p1, xpad)

    s1 = _agg16(hp0, src2d, dst2d)
    hp1 = _tc_layer(s1[0], s1[1], hp0, p0, p1, W1p,
                    b1.reshape(1, 16), False)

    s2 = _agg16(hp1, src2d, dst2d)
    hp2 = _tc_layer(s2[0], s2[1], hp1, p0, p1, W2,
                    b2.reshape(1, 32), False)

    s3 = _agg32(hp2, src2d, dst2d)
    hp3a, hp3b = _tc_layer(s3[0], s3[1], hp2, p0, p1, W3,
                           b3.reshape(1, 64), True)

    s4a, s4b = _agg32f(hp3a, hp3b, src2d, dst2d)
    rows = _tc_layer4(s4a, s4b, hp3a, hp3b, p0, p1, W4,
                      b4.reshape(1, 64), Wlp, blp.reshape(1, 16))

    r = _readout(rows, batch2d)
    return _tc_final(r)
